# Initial kernel scaffold; baseline (speedup 1.0000x reference)
import jax
import jax.numpy as jnp
from jax.experimental import pallas as pl


def kernel(tcr, tcr_len, pmhc, pmhc_len, embed, W1, b1, W2, b2):
    raise NotImplementedError("write your pallas kernel here")


# speedup vs baseline: 30.1301x; 30.1301x over previous
import jax
import jax.numpy as jnp
from jax.experimental import pallas as pl

B = 16384
LT = 50
LP = 200
V = 20
D = 64
H = 128
BLK = 512


def _body(tcr_ref, lt_ref, pmhc_ref, lp_ref, embed_ref, w1_ref, b1_ref,
          w2_ref, b2_ref, out_ref):
    embed = embed_ref[:, :]
    w1 = w1_ref[:, :]
    e1a = jnp.dot(embed, w1[:, :D].T, preferred_element_type=jnp.float32)
    e1b = jnp.dot(embed, w1[:, D:].T, preferred_element_type=jnp.float32)

    iota = jax.lax.broadcasted_iota(jnp.int32, (1, 1, V), 2)
    tcr = tcr_ref[:, :]
    pmhc = pmhc_ref[:, :]
    ct = jnp.sum((tcr[:, :, None] == iota).astype(jnp.float32), axis=1)
    cp = jnp.sum((pmhc[:, :, None] == iota).astype(jnp.float32), axis=1)

    inv_lt = 1.0 / lt_ref[:, :]
    inv_lp = 1.0 / lp_ref[:, :]
    h = (jnp.dot(ct, e1a, preferred_element_type=jnp.float32) * inv_lt
         + jnp.dot(cp, e1b, preferred_element_type=jnp.float32) * inv_lp
         + b1_ref[:, :])
    h = jnp.maximum(h, 0.0)
    out_ref[:, :] = (jnp.sum(h * w2_ref[:, :], axis=1, keepdims=True)
                     + b2_ref[:, :])


def kernel(tcr, tcr_len, pmhc, pmhc_len, embed, W1, b1, W2, b2):
    grid = (B // BLK,)
    out = pl.pallas_call(
        _body,
        grid=grid,
        in_specs=[
            pl.BlockSpec((BLK, LT), lambda i: (i, 0)),
            pl.BlockSpec((BLK, 1), lambda i: (i, 0)),
            pl.BlockSpec((BLK, LP), lambda i: (i, 0)),
            pl.BlockSpec((BLK, 1), lambda i: (i, 0)),
            pl.BlockSpec((V, D), lambda i: (0, 0)),
            pl.BlockSpec((H, 2 * D), lambda i: (0, 0)),
            pl.BlockSpec((1, H), lambda i: (0, 0)),
            pl.BlockSpec((1, H), lambda i: (0, 0)),
            pl.BlockSpec((1, 1), lambda i: (0, 0)),
        ],
        out_specs=pl.BlockSpec((BLK, 1), lambda i: (i, 0)),
        out_shape=jax.ShapeDtypeStruct((B, 1), jnp.float32),
    )(tcr, tcr_len.reshape(B, 1), pmhc, pmhc_len.reshape(B, 1),
      embed, W1, b1.reshape(1, H), W2, b2.reshape(1, 1))
    return out[:, 0]


# speedup vs baseline: 202.1577x; 6.7095x over previous
import jax
import jax.numpy as jnp
from jax.experimental import pallas as pl

B = 16384
LT = 50
LP = 200
V = 20
D = 64
H = 128
CB = 2048


def _counts_t(tok_ref):
    tok = tok_ref[:, :]
    rows = []
    for v in range(V):
        m = (tok == jnp.bfloat16(v)).astype(jnp.bfloat16)
        rows.append(jnp.sum(m, axis=0, keepdims=True))
    return jnp.concatenate(rows, axis=0).astype(jnp.float32)


def _body(tcr_ref, lt_ref, pmhc_ref, lp_ref, embed_ref, w1_ref, b1_ref,
          w2_ref, b2_ref, out_ref):
    embed = embed_ref[:, :]
    w1 = w1_ref[:, :]
    dn = (((1,), (1,)), ((), ()))
    e1a = jax.lax.dot_general(embed, w1[:, :D], dn,
                              preferred_element_type=jnp.float32)
    e1b = jax.lax.dot_general(embed, w1[:, D:], dn,
                              preferred_element_type=jnp.float32)

    ct = _counts_t(tcr_ref) * (1.0 / lt_ref[:, :])
    cp = _counts_t(pmhc_ref) * (1.0 / lp_ref[:, :])

    dnt = (((0,), (0,)), ((), ()))
    h = (jax.lax.dot_general(ct, e1a, dnt, preferred_element_type=jnp.float32)
         + jax.lax.dot_general(cp, e1b, dnt,
                               preferred_element_type=jnp.float32)
         + b1_ref[:, :])
    h = jnp.maximum(h, 0.0)
    out_ref[:, :] = (jnp.sum(h * w2_ref[:, :], axis=1, keepdims=True)
                     + b2_ref[:, :])


def kernel(tcr, tcr_len, pmhc, pmhc_len, embed, W1, b1, W2, b2):
    tcr_t = tcr.T.astype(jnp.bfloat16)
    pmhc_t = pmhc.T.astype(jnp.bfloat16)
    grid = (B // CB,)
    out = pl.pallas_call(
        _body,
        grid=grid,
        in_specs=[
            pl.BlockSpec((LT, CB), lambda i: (0, i)),
            pl.BlockSpec((1, CB), lambda i: (0, i)),
            pl.BlockSpec((LP, CB), lambda i: (0, i)),
            pl.BlockSpec((1, CB), lambda i: (0, i)),
            pl.BlockSpec((V, D), lambda i: (0, 0)),
            pl.BlockSpec((H, 2 * D), lambda i: (0, 0)),
            pl.BlockSpec((1, H), lambda i: (0, 0)),
            pl.BlockSpec((1, H), lambda i: (0, 0)),
            pl.BlockSpec((1, 1), lambda i: (0, 0)),
        ],
        out_specs=pl.BlockSpec((CB, 1), lambda i: (i, 0)),
        out_shape=jax.ShapeDtypeStruct((B, 1), jnp.float32),
    )(tcr_t, tcr_len.reshape(1, B), pmhc_t, pmhc_len.reshape(1, B),
      embed, W1, b1.reshape(1, H), W2, b2.reshape(1, 1))
    return out[:, 0]


# speedup vs baseline: 334.0324x; 1.6523x over previous
import jax
import jax.numpy as jnp
from jax.experimental import pallas as pl

B = 16384
LT = 50
LP = 200
V = 20
D = 64
H = 128
CB = 2048
LTP = 64
LPP = 208


def _counts_t(tok_ref, lp):
    tok = tok_ref[:, :]
    ntile = lp // 16
    one = jnp.ones((), jnp.bfloat16)
    zero = jnp.zeros((), jnp.bfloat16)
    rows = []
    for v in range(V):
        m = jnp.where(tok == jnp.bfloat16(v), one, zero)
        m3 = m.reshape(ntile, 16, CB)
        acc = m3[0]
        for t in range(1, ntile):
            acc = acc + m3[t]
        rows.append(jnp.sum(acc.astype(jnp.float32), axis=0, keepdims=True))
    return jnp.concatenate(rows, axis=0)


def _body(tcr_ref, lt_ref, pmhc_ref, lp_ref, embed_ref, w1_ref, b1_ref,
          w2_ref, b2_ref, out_ref):
    embed = embed_ref[:, :]
    w1 = w1_ref[:, :]
    dn = (((1,), (1,)), ((), ()))
    e1a = jax.lax.dot_general(embed, w1[:, :D], dn,
                              preferred_element_type=jnp.float32)
    e1b = jax.lax.dot_general(embed, w1[:, D:], dn,
                              preferred_element_type=jnp.float32)

    ct = _counts_t(tcr_ref, LTP) * (1.0 / lt_ref[:, :])
    cp = _counts_t(pmhc_ref, LPP) * (1.0 / lp_ref[:, :])

    dnt = (((0,), (0,)), ((), ()))
    h = (jax.lax.dot_general(ct, e1a, dnt, preferred_element_type=jnp.float32)
         + jax.lax.dot_general(cp, e1b, dnt,
                               preferred_element_type=jnp.float32)
         + b1_ref[:, :])
    h = jnp.maximum(h, 0.0)
    out_ref[:, :] = (jnp.sum(h * w2_ref[:, :], axis=1, keepdims=True)
                     + b2_ref[:, :])


def kernel(tcr, tcr_len, pmhc, pmhc_len, embed, W1, b1, W2, b2):
    tcr_t = jnp.pad(tcr.T.astype(jnp.bfloat16), ((0, LTP - LT), (0, 0)),
                    constant_values=jnp.bfloat16(255))
    pmhc_t = jnp.pad(pmhc.T.astype(jnp.bfloat16), ((0, LPP - LP), (0, 0)),
                     constant_values=jnp.bfloat16(255))
    grid = (B // CB,)
    out = pl.pallas_call(
        _body,
        grid=grid,
        in_specs=[
            pl.BlockSpec((LTP, CB), lambda i: (0, i)),
            pl.BlockSpec((1, CB), lambda i: (0, i)),
            pl.BlockSpec((LPP, CB), lambda i: (0, i)),
            pl.BlockSpec((1, CB), lambda i: (0, i)),
            pl.BlockSpec((V, D), lambda i: (0, 0)),
            pl.BlockSpec((H, 2 * D), lambda i: (0, 0)),
            pl.BlockSpec((1, H), lambda i: (0, 0)),
            pl.BlockSpec((1, H), lambda i: (0, 0)),
            pl.BlockSpec((1, 1), lambda i: (0, 0)),
        ],
        out_specs=pl.BlockSpec((CB, 1), lambda i: (i, 0)),
        out_shape=jax.ShapeDtypeStruct((B, 1), jnp.float32),
    )(tcr_t, tcr_len.reshape(1, B), pmhc_t, pmhc_len.reshape(1, B),
      embed, W1, b1.reshape(1, H), W2, b2.reshape(1, 1))
    return out[:, 0]
